# baseline (speedup 1.0000x reference)
import math

import jax
import jax.numpy as jnp
from jax import lax
from jax.experimental import pallas as pl
from jax.experimental.pallas import tpu as pltpu
from jax.experimental.pallas import tpu_sc as plsc

N = 10000
E = 160000
D = 256
H = 8
DH = 32
FF = 4 * D

NC = 2
NS = 16
HALF = D // 2
HPC = H // NC
NP = 10240
NDR = NP * HPC // 128
EPW = E // NS
CHUNK = 40
NCHUNK = EPW // CHUNK
RPS = NP // NS
DRPS = 32
INV_SQRT_DH = 1.0 / math.sqrt(DH)

BN = 1000


def _tc1_body(h_ref, wfcT_ref, g_ref, b_ref, hn_ref, ft_ref):
    x = h_ref[...]
    m = jnp.mean(x, axis=1, keepdims=True)
    v = jnp.mean((x - m) ** 2, axis=1, keepdims=True)
    hn = (x - m) / jnp.sqrt(v + 1e-5) * g_ref[...] + b_ref[...]
    hn_ref[...] = hn
    ft_ref[...] = jnp.dot(hn, wfcT_ref[...], preferred_element_type=jnp.float32)


def _tc1(h, wfcT, g, b):
    grid = N // BN
    return pl.pallas_call(
        _tc1_body,
        grid=(grid,),
        in_specs=[
            pl.BlockSpec((BN, D), lambda i: (i, 0)),
            pl.BlockSpec((D, D), lambda i: (0, 0)),
            pl.BlockSpec((1, D), lambda i: (0, 0)),
            pl.BlockSpec((1, D), lambda i: (0, 0)),
        ],
        out_specs=[
            pl.BlockSpec((BN, D), lambda i: (i, 0)),
            pl.BlockSpec((BN, D), lambda i: (i, 0)),
        ],
        out_shape=[
            jax.ShapeDtypeStruct((N, D), jnp.float32),
            jax.ShapeDtypeStruct((N, D), jnp.float32),
        ],
    )(h, wfcT, g, b)


def _edge_body(ft2_hbm, src2_hbm, dst3_hbm, zn_hbm, out_hbm, den_hbm,
               sidx0, sidx1, didx0, didx1, scr0, scr1, sc20, sc21,
               smi0, smi1, srows0, srows1, drows0, drows1, msg0, msg1,
               den0, den1, accum, den_sh,
               ssem_s, ssem_r, gs0, gs1, gd0, gd1, scm0, scm1, scd0, scd1):
    c = lax.axis_index("c")
    s = lax.axis_index("s")

    sidx = (sidx0, sidx1)
    didx = (didx0, didx1)
    scr = (scr0, scr1)
    sc2 = (sc20, sc21)
    smi = (smi0, smi1)
    srows = (srows0, srows1)
    drows = (drows0, drows1)
    msg = (msg0, msg1)
    den_row = (den0, den1)
    gsem1 = (gs0, gs1)
    gsem2 = (gd0, gd1)
    scm = (scm0, scm1)
    scd = (scd0, scd1)

    rbase = s * RPS
    pltpu.sync_copy(zn_hbm.at[pl.ds(rbase, RPS)], accum.at[pl.ds(rbase, RPS)])
    dbase = s * DRPS

    @pl.when(s < NDR // DRPS)
    def _init_den():
        pltpu.sync_copy(zn_hbm.at[pl.ds(dbase, DRPS)],
                        den_sh.at[pl.ds(dbase, DRPS)])
    plsc.subcore_barrier()

    iota = lax.iota(jnp.int32, 16)
    lane_head = iota < HPC
    cpad = jnp.where(iota >= 8, c, 0)
    zero16 = jnp.zeros((16,), jnp.float32)
    ebase = s * EPW

    def _stage_start(q, b):
        off = ebase + q * CHUNK
        pltpu.make_async_copy(src2_hbm.at[pl.ds(off, CHUNK)],
                              sidx[b], ssem_s).start()
        pltpu.make_async_copy(dst3_hbm.at[s, q], scr[b], ssem_r).start()

    def _stage_wait(b):
        pltpu.make_async_copy(src2_hbm.at[pl.ds(0, CHUNK)],
                              sidx[b], ssem_s).wait()
        pltpu.make_async_copy(dst3_hbm.at[s, 0], scr[b], ssem_r).wait()

    def _transform(b):
        for off in (0, 16):
            sidx[b][pl.ds(off, 16)] = sidx[b][pl.ds(off, 16)] + c
            dv = scr[b][0, pl.ds(off, 16)]
            didx[b][pl.ds(off, 16)] = dv * 2 + c
            sc2[b][0, pl.ds(off, 16)] = lax.shift_right_logical(dv, 5)
            smi[b][0, pl.ds(off, 16)] = dv
        sidx[b][pl.ds(24, 16)] = sidx[b][pl.ds(24, 16)] + cpad
        dv = scr[b][0, pl.ds(24, 16)]
        didx[b][pl.ds(24, 16)] = dv * 2 + c
        sc2[b][0, pl.ds(24, 16)] = lax.shift_right_logical(dv, 5)
        smi[b][0, pl.ds(24, 16)] = dv

    def _gather_start(b):
        pltpu.make_async_copy(ft2_hbm.at[sidx[b]], srows[b], gsem1[b]).start()
        pltpu.make_async_copy(ft2_hbm.at[didx[b]], drows[b], gsem2[b]).start()

    def _gather_wait(b):
        pltpu.make_async_copy(ft2_hbm.at[sidx[b]], srows[b], gsem1[b]).wait()
        pltpu.make_async_copy(ft2_hbm.at[didx[b]], drows[b], gsem2[b]).wait()

    def _scatter_start(b):
        pltpu.make_async_copy(msg[b], accum.at[smi[b].at[0]],
                              scm[b]).start(add=True)
        pltpu.make_async_copy(den_row[b], den_sh.at[sc2[b].at[0]],
                              scd[b]).start(add=True)

    def _scatter_wait(b):
        pltpu.make_async_copy(msg[b], accum.at[smi[b].at[0]],
                              scm[b]).wait()
        pltpu.make_async_copy(den_row[b], den_sh.at[sc2[b].at[0]],
                              scd[b]).wait()

    def _compute(b):
        @plsc.parallel_loop(0, CHUNK, unroll=8)
        def _edge(i):
            tail = zero16
            for h2 in range(HPC):
                s0 = srows[b][i, pl.ds(h2 * 32, 16)]
                s1 = srows[b][i, pl.ds(h2 * 32 + 16, 16)]
                d0 = drows[b][i, pl.ds(h2 * 32, 16)]
                d1 = drows[b][i, pl.ds(h2 * 32 + 16, 16)]
                p = s0 * d0 + s1 * d1
                e = jnp.sum(p) * INV_SQRT_DH
                ex = jnp.exp(jnp.broadcast_to(e, (16,)))
                msg[b][i, pl.ds(h2 * 32, 16)] = s0 * ex
                msg[b][i, pl.ds(h2 * 32 + 16, 16)] = s1 * ex
                tail = tail + jnp.where(iota == h2, ex, 0.0)
            for v in range(8):
                den_row[b][i, pl.ds(v * 16, 16)] = zero16
            dstv = plsc.load_gather(
                smi[b], [jnp.zeros((16,), jnp.int32),
                         jnp.full((16,), i, jnp.int32)])
            pcol = (dstv & 31) * HPC + iota
            plsc.store_scatter(den_row[b],
                               [jnp.full((16,), i, jnp.int32), pcol],
                               tail, mask=lane_head)

    _stage_start(0, 0)
    _stage_wait(0)
    _transform(0)
    _gather_start(0)
    _stage_start(1, 1)

    def _pair(jo, _):
        for b in (0, 1):
            q = 2 * jo + b

            @pl.when(q >= 1)
            def _w():
                _scatter_wait(b ^ 1)

            @pl.when(q + 1 < NCHUNK)
            def _t():
                _stage_wait(b ^ 1)
                _transform(b ^ 1)
                _gather_start(b ^ 1)

            _gather_wait(b)

            @pl.when(q + 2 < NCHUNK)
            def _s():
                _stage_start(q + 2, b)

            _compute(b)
            _scatter_start(b)
        return 0

    lax.fori_loop(0, NCHUNK // 2, _pair, 0)
    _scatter_wait((NCHUNK - 1) % 2)

    plsc.subcore_barrier()
    pltpu.sync_copy(accum.at[pl.ds(rbase, RPS)],
                    out_hbm.at[c, pl.ds(rbase, RPS)])
    @pl.when(s < NDR // DRPS)
    def _flush_den():
        pltpu.sync_copy(den_sh.at[pl.ds(dbase, DRPS)],
                        den_hbm.at[c, pl.ds(dbase, DRPS)])


def _edge_sc(ft2, src2, dst3, zn):
    mesh = plsc.VectorSubcoreMesh(core_axis_name="c", subcore_axis_name="s")
    return pl.kernel(
        _edge_body,
        out_type=[
            jax.ShapeDtypeStruct((NC, NP, HALF), jnp.float32),
            jax.ShapeDtypeStruct((NC, NDR, HALF), jnp.float32),
        ],
        mesh=mesh,
        compiler_params=pltpu.CompilerParams(needs_layout_passes=False),
        scratch_types=(
            [pltpu.VMEM((CHUNK,), jnp.int32)] * 4
            + [pltpu.VMEM((1, CHUNK), jnp.int32)] * 6
            + [pltpu.VMEM((CHUNK, HALF), jnp.float32)] * 8
            + [pltpu.VMEM_SHARED((NP, HALF), jnp.float32),
               pltpu.VMEM_SHARED((NDR, HALF), jnp.float32)]
            + [pltpu.SemaphoreType.DMA] * 10
        ),
    )(ft2, src2, dst3, zn)


def _tc2_body(a0_ref, a1_ref, d0_ref, d1_ref, hn_ref, s4_ref, wredT_ref,
              bred_ref, g2_ref, b2g_ref, w1T_ref, b1_ref, w2T_ref, b2_ref,
              out_ref):
    s4 = s4_ref[...]
    bden0 = jnp.dot(d0_ref[...], s4, preferred_element_type=jnp.float32)
    bden1 = jnp.dot(d1_ref[...], s4, preferred_element_type=jnp.float32)
    att0 = a0_ref[...] / (bden0 + 1e-9)
    att1 = a1_ref[...] / (bden1 + 1e-9)
    att = jnp.concatenate([att0, att1], axis=1)
    out = (jnp.dot(att, wredT_ref[...], preferred_element_type=jnp.float32)
           + bred_ref[...] + hn_ref[...])
    m = jnp.mean(out, axis=1, keepdims=True)
    v = jnp.mean((out - m) ** 2, axis=1, keepdims=True)
    hs2 = (out - m) / jnp.sqrt(v + 1e-5) * g2_ref[...] + b2g_ref[...]
    x = jnp.dot(hs2.astype(jnp.bfloat16), w1T_ref[...],
                preferred_element_type=jnp.float32) + b1_ref[...]
    x = jnp.where(x > 0, x, jnp.exp(jnp.where(x > 0, 0.0, x)) - 1.0)
    y = jnp.dot(x.astype(jnp.bfloat16), w2T_ref[...],
                preferred_element_type=jnp.float32) + b2_ref[...]
    y = jnp.where(y > 0, y, jnp.exp(jnp.where(y > 0, 0.0, y)) - 1.0)
    out_ref[...] = y + hs2


def _tc2(a0, a1, d0, d1, hn, s4, wredT, bred, g2, b2g, w1T, b1, w2T, b2):
    grid = N // BN
    return pl.pallas_call(
        _tc2_body,
        grid=(grid,),
        in_specs=[
            pl.BlockSpec((BN, HALF), lambda i: (i, 0)),
            pl.BlockSpec((BN, HALF), lambda i: (i, 0)),
            pl.BlockSpec((BN, HPC), lambda i: (i, 0)),
            pl.BlockSpec((BN, HPC), lambda i: (i, 0)),
            pl.BlockSpec((BN, D), lambda i: (i, 0)),
            pl.BlockSpec((HPC, HALF), lambda i: (0, 0)),
            pl.BlockSpec((D, D), lambda i: (0, 0)),
            pl.BlockSpec((1, D), lambda i: (0, 0)),
            pl.BlockSpec((1, D), lambda i: (0, 0)),
            pl.BlockSpec((1, D), lambda i: (0, 0)),
            pl.BlockSpec((D, FF), lambda i: (0, 0)),
            pl.BlockSpec((1, FF), lambda i: (0, 0)),
            pl.BlockSpec((FF, D), lambda i: (0, 0)),
            pl.BlockSpec((1, D), lambda i: (0, 0)),
        ],
        out_specs=pl.BlockSpec((BN, D), lambda i: (i, 0)),
        out_shape=jax.ShapeDtypeStruct((N, D), jnp.float32),
    )(a0, a1, d0, d1, hn, s4, wredT, bred, g2, b2g, w1T, b1, w2T, b2)


def kernel(h, edge_index, Wfc, Wred, bred, ln1_g, ln1_b, ln2_g, ln2_b, W1, b1, W2, b2):
    src = edge_index[0]
    dst = edge_index[1]

    hn, ft = _tc1(h, Wfc.T, ln1_g.reshape(1, D), ln1_b.reshape(1, D))

    ft2 = ft.reshape(2 * N, HALF)
    src2 = src * 2
    dst3 = dst.reshape(NS, NCHUNK, 1, CHUNK)
    zn = jnp.zeros((NP, HALF), jnp.float32)

    acc, den = _edge_sc(ft2, src2, dst3, zn)
    dp = den.reshape(NC, NP, HPC)

    s4 = jnp.repeat(jnp.eye(HPC, dtype=jnp.float32), DH, axis=1)

    out = _tc2(acc[0], acc[1], dp[0], dp[1], hn, s4, Wred.T,
               bred.reshape(1, D), ln2_g.reshape(1, D), ln2_b.reshape(1, D),
               W1.T.astype(jnp.bfloat16), b1.reshape(1, FF),
               W2.T.astype(jnp.bfloat16), b2.reshape(1, D))
    return out


# speedup vs baseline: 1.0517x; 1.0517x over previous
import math

import jax
import jax.numpy as jnp
from jax import lax
from jax.experimental import pallas as pl
from jax.experimental.pallas import tpu as pltpu
from jax.experimental.pallas import tpu_sc as plsc

N = 10000
E = 160000
D = 256
H = 8
DH = 32
FF = 4 * D

NC = 2
NS = 16
HALF = D // 2
HPC = H // NC
NP = 10240
NDR = NP * HPC // 128
EPW = E // NS
CHUNK = 40
NCHUNK = EPW // CHUNK
RPS = NP // NS
DRPS = 32
INV_SQRT_DH = 1.0 / math.sqrt(DH)

BN = 1000


def _tc1_body(h_ref, wfcT_ref, g_ref, b_ref, hn_ref, ft_ref):
    x = h_ref[...]
    m = jnp.mean(x, axis=1, keepdims=True)
    v = jnp.mean((x - m) ** 2, axis=1, keepdims=True)
    hn = (x - m) / jnp.sqrt(v + 1e-5) * g_ref[...] + b_ref[...]
    hn_ref[...] = hn
    ft_ref[...] = jnp.dot(hn, wfcT_ref[...], preferred_element_type=jnp.float32)


def _tc1(h, wfcT, g, b):
    grid = N // BN
    return pl.pallas_call(
        _tc1_body,
        grid=(grid,),
        in_specs=[
            pl.BlockSpec((BN, D), lambda i: (i, 0)),
            pl.BlockSpec((D, D), lambda i: (0, 0)),
            pl.BlockSpec((1, D), lambda i: (0, 0)),
            pl.BlockSpec((1, D), lambda i: (0, 0)),
        ],
        out_specs=[
            pl.BlockSpec((BN, D), lambda i: (i, 0)),
            pl.BlockSpec((BN, D), lambda i: (i, 0)),
        ],
        out_shape=[
            jax.ShapeDtypeStruct((N, D), jnp.float32),
            jax.ShapeDtypeStruct((N, D), jnp.float32),
        ],
    )(h, wfcT, g, b)


def _edge_body(ft2_hbm, src2_hbm, dst3_hbm, zn_hbm, out_hbm, den_hbm,
               sidx0, sidx1, didx0, didx1, scr0, scr1, sc20, sc21,
               smi0, smi1, srows0, srows1, drows0, drows1, msg0, msg1,
               den0, den1, accum, den_sh,
               ssem_s, ssem_r, gs0, gs1, gd0, gd1, scm0, scm1, scd0, scd1):
    c = lax.axis_index("c")
    s = lax.axis_index("s")

    sidx = (sidx0, sidx1)
    didx = (didx0, didx1)
    scr = (scr0, scr1)
    sc2 = (sc20, sc21)
    smi = (smi0, smi1)
    srows = (srows0, srows1)
    drows = (drows0, drows1)
    msg = (msg0, msg1)
    den_row = (den0, den1)
    gsem1 = (gs0, gs1)
    gsem2 = (gd0, gd1)
    scm = (scm0, scm1)
    scd = (scd0, scd1)

    rbase = s * RPS
    pltpu.sync_copy(zn_hbm.at[pl.ds(rbase, RPS)], accum.at[pl.ds(rbase, RPS)])
    dbase = s * DRPS

    @pl.when(s < NDR // DRPS)
    def _init_den():
        pltpu.sync_copy(zn_hbm.at[pl.ds(dbase, DRPS)],
                        den_sh.at[pl.ds(dbase, DRPS)])
    plsc.subcore_barrier()

    iota = lax.iota(jnp.int32, 16)
    lane_head = iota < HPC
    cpad = jnp.where(iota >= 8, c, 0)
    zero16 = jnp.zeros((16,), jnp.float32)
    ebase = s * EPW

    def _stage_start(q, b):
        off = ebase + q * CHUNK
        pltpu.make_async_copy(src2_hbm.at[pl.ds(off, CHUNK)],
                              sidx[b], ssem_s).start()
        pltpu.make_async_copy(dst3_hbm.at[s, q], scr[b], ssem_r).start()

    def _stage_wait(b):
        pltpu.make_async_copy(src2_hbm.at[pl.ds(0, CHUNK)],
                              sidx[b], ssem_s).wait()
        pltpu.make_async_copy(dst3_hbm.at[s, 0], scr[b], ssem_r).wait()

    def _transform(b):
        for off in (0, 16):
            sidx[b][pl.ds(off, 16)] = sidx[b][pl.ds(off, 16)] + c
            dv = scr[b][0, pl.ds(off, 16)]
            didx[b][pl.ds(off, 16)] = dv * 2 + c
            sc2[b][0, pl.ds(off, 16)] = lax.shift_right_logical(dv, 5)
            smi[b][0, pl.ds(off, 16)] = dv
        sidx[b][pl.ds(24, 16)] = sidx[b][pl.ds(24, 16)] + cpad
        dv = scr[b][0, pl.ds(24, 16)]
        didx[b][pl.ds(24, 16)] = dv * 2 + c
        sc2[b][0, pl.ds(24, 16)] = lax.shift_right_logical(dv, 5)
        smi[b][0, pl.ds(24, 16)] = dv

    def _gather_start(b):
        pltpu.make_async_copy(ft2_hbm.at[sidx[b]], srows[b], gsem1[b]).start()
        pltpu.make_async_copy(ft2_hbm.at[didx[b]], drows[b], gsem2[b]).start()

    def _gather_wait(b):
        pltpu.make_async_copy(ft2_hbm.at[sidx[b]], srows[b], gsem1[b]).wait()
        pltpu.make_async_copy(ft2_hbm.at[didx[b]], drows[b], gsem2[b]).wait()

    def _scatter_start(b):
        pltpu.make_async_copy(msg[b], accum.at[smi[b].at[0]],
                              scm[b]).start(add=True)
        pltpu.make_async_copy(den_row[b], den_sh.at[sc2[b].at[0]],
                              scd[b]).start(add=True)

    def _scatter_wait(b):
        pltpu.make_async_copy(msg[b], accum.at[smi[b].at[0]],
                              scm[b]).wait()
        pltpu.make_async_copy(den_row[b], den_sh.at[sc2[b].at[0]],
                              scd[b]).wait()

    def _compute(b):
        @plsc.parallel_loop(0, CHUNK, unroll=8)
        def _edge(i):
            tail = zero16
            for h2 in range(HPC):
                s0 = srows[b][i, pl.ds(h2 * 32, 16)]
                s1 = srows[b][i, pl.ds(h2 * 32 + 16, 16)]
                d0 = drows[b][i, pl.ds(h2 * 32, 16)]
                d1 = drows[b][i, pl.ds(h2 * 32 + 16, 16)]
                p = s0 * d0 + s1 * d1
                e = jnp.sum(p) * INV_SQRT_DH
                ex = jnp.exp(jnp.broadcast_to(e, (16,)))
                msg[b][i, pl.ds(h2 * 32, 16)] = s0 * ex
                msg[b][i, pl.ds(h2 * 32 + 16, 16)] = s1 * ex
                tail = tail + jnp.where(iota == h2, ex, 0.0)
            for v in range(8):
                den_row[b][i, pl.ds(v * 16, 16)] = zero16
            dstv = plsc.load_gather(
                smi[b], [jnp.zeros((16,), jnp.int32),
                         jnp.full((16,), i, jnp.int32)])
            pcol = (dstv & 31) * HPC + iota
            plsc.store_scatter(den_row[b],
                               [jnp.full((16,), i, jnp.int32), pcol],
                               tail, mask=lane_head)

    _stage_start(0, 0)
    _stage_wait(0)
    _transform(0)
    _gather_start(0)
    _stage_start(1, 1)

    def _pair(jo, _):
        for b in (0, 1):
            q = 2 * jo + b

            @pl.when(q >= 1)
            def _w():
                _scatter_wait(b ^ 1)

            @pl.when(q + 1 < NCHUNK)
            def _t():
                _stage_wait(b ^ 1)
                _transform(b ^ 1)
                _gather_start(b ^ 1)

            _gather_wait(b)

            @pl.when(q + 2 < NCHUNK)
            def _s():
                _stage_start(q + 2, b)

            _compute(b)
            _scatter_start(b)
        return 0

    lax.fori_loop(0, NCHUNK // 2, _pair, 0)
    _scatter_wait((NCHUNK - 1) % 2)

    plsc.subcore_barrier()
    pltpu.sync_copy(accum.at[pl.ds(rbase, RPS)],
                    out_hbm.at[c, pl.ds(rbase, RPS)])
    @pl.when(s < NDR // DRPS)
    def _flush_den():
        pltpu.sync_copy(den_sh.at[pl.ds(dbase, DRPS)],
                        den_hbm.at[c, pl.ds(dbase, DRPS)])


def _edge_sc(ft2, src2, dst3, zn):
    mesh = plsc.VectorSubcoreMesh(core_axis_name="c", subcore_axis_name="s")
    return pl.kernel(
        _edge_body,
        out_type=[
            jax.ShapeDtypeStruct((NC, NP, HALF), jnp.float32),
            jax.ShapeDtypeStruct((NC, NDR, HALF), jnp.float32),
        ],
        mesh=mesh,
        compiler_params=pltpu.CompilerParams(needs_layout_passes=False),
        scratch_types=(
            [pltpu.VMEM((CHUNK,), jnp.int32)] * 4
            + [pltpu.VMEM((1, CHUNK), jnp.int32)] * 6
            + [pltpu.VMEM((CHUNK, HALF), jnp.float32)] * 8
            + [pltpu.VMEM_SHARED((NP, HALF), jnp.float32),
               pltpu.VMEM_SHARED((NDR, HALF), jnp.float32)]
            + [pltpu.SemaphoreType.DMA] * 10
        ),
    )(ft2, src2, dst3, zn)


def _tc2_body(a0_ref, a1_ref, d0_ref, d1_ref, hn_ref, s4_ref, wredT_ref,
              bred_ref, g2_ref, b2g_ref, w1T_ref, b1_ref, w2T_ref, b2_ref,
              out_ref):
    s4 = s4_ref[...]
    bden0 = jnp.dot(d0_ref[0], s4, preferred_element_type=jnp.float32)
    bden1 = jnp.dot(d1_ref[0], s4, preferred_element_type=jnp.float32)
    att0 = a0_ref[0] / (bden0 + 1e-9)
    att1 = a1_ref[0] / (bden1 + 1e-9)
    att = jnp.concatenate([att0, att1], axis=1)
    out = (jnp.dot(att, wredT_ref[...], preferred_element_type=jnp.float32)
           + bred_ref[...] + hn_ref[...])
    m = jnp.mean(out, axis=1, keepdims=True)
    v = jnp.mean((out - m) ** 2, axis=1, keepdims=True)
    hs2 = (out - m) / jnp.sqrt(v + 1e-5) * g2_ref[...] + b2g_ref[...]
    x = jnp.dot(hs2, w1T_ref[...], preferred_element_type=jnp.float32) + b1_ref[...]
    x = jnp.where(x > 0, x, jnp.exp(jnp.where(x > 0, 0.0, x)) - 1.0)
    y = jnp.dot(x, w2T_ref[...], preferred_element_type=jnp.float32) + b2_ref[...]
    y = jnp.where(y > 0, y, jnp.exp(jnp.where(y > 0, 0.0, y)) - 1.0)
    out_ref[...] = y + hs2


def _tc2(a0, a1, d0, d1, hn, s4, wredT, bred, g2, b2g, w1T, b1, w2T, b2):
    grid = N // BN
    return pl.pallas_call(
        _tc2_body,
        grid=(grid,),
        in_specs=[
            pl.BlockSpec((1, BN, HALF), lambda i: (0, i, 0)),
            pl.BlockSpec((1, BN, HALF), lambda i: (1, i, 0)),
            pl.BlockSpec((1, BN, HPC), lambda i: (0, i, 0)),
            pl.BlockSpec((1, BN, HPC), lambda i: (1, i, 0)),
            pl.BlockSpec((BN, D), lambda i: (i, 0)),
            pl.BlockSpec((HPC, HALF), lambda i: (0, 0)),
            pl.BlockSpec((D, D), lambda i: (0, 0)),
            pl.BlockSpec((1, D), lambda i: (0, 0)),
            pl.BlockSpec((1, D), lambda i: (0, 0)),
            pl.BlockSpec((1, D), lambda i: (0, 0)),
            pl.BlockSpec((D, FF), lambda i: (0, 0)),
            pl.BlockSpec((1, FF), lambda i: (0, 0)),
            pl.BlockSpec((FF, D), lambda i: (0, 0)),
            pl.BlockSpec((1, D), lambda i: (0, 0)),
        ],
        out_specs=pl.BlockSpec((BN, D), lambda i: (i, 0)),
        out_shape=jax.ShapeDtypeStruct((N, D), jnp.float32),
    )(a0, a1, d0, d1, hn, s4, wredT, bred, g2, b2g, w1T, b1, w2T, b2)


def kernel(h, edge_index, Wfc, Wred, bred, ln1_g, ln1_b, ln2_g, ln2_b, W1, b1, W2, b2):
    src = edge_index[0]
    dst = edge_index[1]

    hn, ft = _tc1(h, Wfc.T, ln1_g.reshape(1, D), ln1_b.reshape(1, D))

    ft2 = ft.reshape(2 * N, HALF)
    src2 = src * 2
    dst3 = dst.reshape(NS, NCHUNK, 1, CHUNK)
    zn = jnp.zeros((NP, HALF), jnp.float32)

    acc, den = _edge_sc(ft2, src2, dst3, zn)
    dp = den.reshape(NC, NP, HPC)

    s4 = jnp.repeat(jnp.eye(HPC, dtype=jnp.float32), DH, axis=1)

    out = _tc2(acc, acc, dp, dp, hn, s4, Wred.T,
               bred.reshape(1, D), ln2_g.reshape(1, D), ln2_b.reshape(1, D),
               W1.T, b1.reshape(1, FF), W2.T, b2.reshape(1, D))
    return out


# speedup vs baseline: 1.0817x; 1.0285x over previous
import math

import jax
import jax.numpy as jnp
from jax import lax
from jax.experimental import pallas as pl
from jax.experimental.pallas import tpu as pltpu
from jax.experimental.pallas import tpu_sc as plsc

N = 10000
E = 160000
D = 256
H = 8
DH = 32
FF = 4 * D

NC = 2
NS = 16
HALF = D // 2
HPC = H // NC
NP = 10240
NDR = NP * HPC // 128
EPW = E // NS
CHUNK = 40
NCHUNK = EPW // CHUNK
RPS = NP // NS
DRPS = 32
INV_SQRT_DH = 1.0 / math.sqrt(DH)

BN = 1000


def _tc1_body(h_ref, wfcT_ref, g_ref, b_ref, hn_ref, ft_ref):
    x = h_ref[...]
    m = jnp.mean(x, axis=1, keepdims=True)
    v = jnp.mean((x - m) ** 2, axis=1, keepdims=True)
    hn = (x - m) / jnp.sqrt(v + 1e-5) * g_ref[...] + b_ref[...]
    hn_ref[...] = hn
    ft = jnp.dot(hn, wfcT_ref[...], preferred_element_type=jnp.float32)
    ft_ref[...] = ft.reshape(2 * BN, HALF)


def _tc1(h, wfcT, g, b):
    grid = N // BN
    return pl.pallas_call(
        _tc1_body,
        grid=(grid,),
        in_specs=[
            pl.BlockSpec((BN, D), lambda i: (i, 0)),
            pl.BlockSpec((D, D), lambda i: (0, 0)),
            pl.BlockSpec((1, D), lambda i: (0, 0)),
            pl.BlockSpec((1, D), lambda i: (0, 0)),
        ],
        out_specs=[
            pl.BlockSpec((BN, D), lambda i: (i, 0)),
            pl.BlockSpec((2 * BN, HALF), lambda i: (i, 0)),
        ],
        out_shape=[
            jax.ShapeDtypeStruct((N, D), jnp.float32),
            jax.ShapeDtypeStruct((2 * N, HALF), jnp.float32),
        ],
    )(h, wfcT, g, b)


def _edge_body(ft2_hbm, src2_hbm, dst3_hbm, zn_hbm, out_hbm, den_hbm,
               sidx0, sidx1, didx0, didx1, scr0, scr1, sc20, sc21,
               smi0, smi1, srows0, srows1, drows0, drows1, msg0, msg1,
               den0, den1, accum, den_sh,
               ssem_s, ssem_r, gs0, gs1, gd0, gd1, scm0, scm1, scd0, scd1):
    c = lax.axis_index("c")
    s = lax.axis_index("s")

    sidx = (sidx0, sidx1)
    didx = (didx0, didx1)
    scr = (scr0, scr1)
    sc2 = (sc20, sc21)
    smi = (smi0, smi1)
    srows = (srows0, srows1)
    drows = (drows0, drows1)
    msg = (msg0, msg1)
    den_row = (den0, den1)
    gsem1 = (gs0, gs1)
    gsem2 = (gd0, gd1)
    scm = (scm0, scm1)
    scd = (scd0, scd1)

    rbase = s * RPS
    pltpu.sync_copy(zn_hbm.at[pl.ds(rbase, RPS)], accum.at[pl.ds(rbase, RPS)])
    dbase = s * DRPS

    @pl.when(s < NDR // DRPS)
    def _init_den():
        pltpu.sync_copy(zn_hbm.at[pl.ds(dbase, DRPS)],
                        den_sh.at[pl.ds(dbase, DRPS)])
    plsc.subcore_barrier()

    iota = lax.iota(jnp.int32, 16)
    lane_head = iota < HPC
    cpad = jnp.where(iota >= 8, c, 0)
    zero16 = jnp.zeros((16,), jnp.float32)
    ebase = s * EPW

    def _stage_start(q, b):
        off = ebase + q * CHUNK
        pltpu.make_async_copy(src2_hbm.at[pl.ds(off, CHUNK)],
                              sidx[b], ssem_s).start()
        pltpu.make_async_copy(dst3_hbm.at[s, q], scr[b], ssem_r).start()

    def _stage_wait(b):
        pltpu.make_async_copy(src2_hbm.at[pl.ds(0, CHUNK)],
                              sidx[b], ssem_s).wait()
        pltpu.make_async_copy(dst3_hbm.at[s, 0], scr[b], ssem_r).wait()

    def _transform(b):
        for off in (0, 16):
            sidx[b][pl.ds(off, 16)] = sidx[b][pl.ds(off, 16)] + c
            dv = scr[b][0, pl.ds(off, 16)]
            didx[b][pl.ds(off, 16)] = dv * 2 + c
            sc2[b][0, pl.ds(off, 16)] = lax.shift_right_logical(dv, 5)
            smi[b][0, pl.ds(off, 16)] = dv
        sidx[b][pl.ds(24, 16)] = sidx[b][pl.ds(24, 16)] + cpad
        dv = scr[b][0, pl.ds(24, 16)]
        didx[b][pl.ds(24, 16)] = dv * 2 + c
        sc2[b][0, pl.ds(24, 16)] = lax.shift_right_logical(dv, 5)
        smi[b][0, pl.ds(24, 16)] = dv

    def _gather_start(b):
        pltpu.make_async_copy(ft2_hbm.at[sidx[b]], srows[b], gsem1[b]).start()
        pltpu.make_async_copy(ft2_hbm.at[didx[b]], drows[b], gsem2[b]).start()

    def _gather_wait(b):
        pltpu.make_async_copy(ft2_hbm.at[sidx[b]], srows[b], gsem1[b]).wait()
        pltpu.make_async_copy(ft2_hbm.at[didx[b]], drows[b], gsem2[b]).wait()

    def _scatter_start(b):
        pltpu.make_async_copy(msg[b], accum.at[smi[b].at[0]],
                              scm[b]).start(add=True)
        pltpu.make_async_copy(den_row[b], den_sh.at[sc2[b].at[0]],
                              scd[b]).start(add=True)

    def _scatter_wait(b):
        pltpu.make_async_copy(msg[b], accum.at[smi[b].at[0]],
                              scm[b]).wait()
        pltpu.make_async_copy(den_row[b], den_sh.at[sc2[b].at[0]],
                              scd[b]).wait()

    def _compute(b):
        @plsc.parallel_loop(0, CHUNK, unroll=8)
        def _edge(i):
            tail = zero16
            for h2 in range(HPC):
                s0 = srows[b][i, pl.ds(h2 * 32, 16)]
                s1 = srows[b][i, pl.ds(h2 * 32 + 16, 16)]
                d0 = drows[b][i, pl.ds(h2 * 32, 16)]
                d1 = drows[b][i, pl.ds(h2 * 32 + 16, 16)]
                p = s0 * d0 + s1 * d1
                e = jnp.sum(p) * INV_SQRT_DH
                ex = jnp.exp(jnp.broadcast_to(e, (16,)))
                msg[b][i, pl.ds(h2 * 32, 16)] = s0 * ex
                msg[b][i, pl.ds(h2 * 32 + 16, 16)] = s1 * ex
                tail = tail + jnp.where(iota == h2, ex, 0.0)
            for v in range(8):
                den_row[b][i, pl.ds(v * 16, 16)] = zero16
            dstv = plsc.load_gather(
                smi[b], [jnp.zeros((16,), jnp.int32),
                         jnp.full((16,), i, jnp.int32)])
            pcol = (dstv & 31) * HPC + iota
            plsc.store_scatter(den_row[b],
                               [jnp.full((16,), i, jnp.int32), pcol],
                               tail, mask=lane_head)

    _stage_start(0, 0)
    _stage_wait(0)
    _transform(0)
    _gather_start(0)
    _stage_start(1, 1)

    def _pair(jo, _):
        for b in (0, 1):
            q = 2 * jo + b

            @pl.when(q >= 1)
            def _w():
                _scatter_wait(b ^ 1)

            @pl.when(q + 1 < NCHUNK)
            def _t():
                _stage_wait(b ^ 1)
                _transform(b ^ 1)
                _gather_start(b ^ 1)

            _gather_wait(b)

            @pl.when(q + 2 < NCHUNK)
            def _s():
                _stage_start(q + 2, b)

            _compute(b)
            _scatter_start(b)
        return 0

    lax.fori_loop(0, NCHUNK // 2, _pair, 0)
    _scatter_wait((NCHUNK - 1) % 2)

    plsc.subcore_barrier()
    pltpu.sync_copy(accum.at[pl.ds(rbase, RPS)],
                    out_hbm.at[c, pl.ds(rbase, RPS)])
    @pl.when(s < NDR // DRPS)
    def _flush_den():
        pltpu.sync_copy(den_sh.at[pl.ds(dbase, DRPS)],
                        den_hbm.at[c, pl.ds(dbase, DRPS)])


def _edge_sc(ft2, src2, dst3, zn):
    mesh = plsc.VectorSubcoreMesh(core_axis_name="c", subcore_axis_name="s")
    return pl.kernel(
        _edge_body,
        out_type=[
            jax.ShapeDtypeStruct((NC, NP, HALF), jnp.float32),
            jax.ShapeDtypeStruct((NC, NDR, HALF), jnp.float32),
        ],
        mesh=mesh,
        compiler_params=pltpu.CompilerParams(needs_layout_passes=False),
        scratch_types=(
            [pltpu.VMEM((CHUNK,), jnp.int32)] * 4
            + [pltpu.VMEM((1, CHUNK), jnp.int32)] * 6
            + [pltpu.VMEM((CHUNK, HALF), jnp.float32)] * 8
            + [pltpu.VMEM_SHARED((NP, HALF), jnp.float32),
               pltpu.VMEM_SHARED((NDR, HALF), jnp.float32)]
            + [pltpu.SemaphoreType.DMA] * 10
        ),
    )(ft2, src2, dst3, zn)


def _tc2_body(a0_ref, a1_ref, d0_ref, d1_ref, hn_ref, s4_ref, wredT_ref,
              bred_ref, g2_ref, b2g_ref, w1T_ref, b1_ref, w2T_ref, b2_ref,
              out_ref):
    s4 = s4_ref[...]
    bden0 = jnp.dot(d0_ref[0], s4, preferred_element_type=jnp.float32)
    bden1 = jnp.dot(d1_ref[0], s4, preferred_element_type=jnp.float32)
    att0 = a0_ref[0] / (bden0 + 1e-9)
    att1 = a1_ref[0] / (bden1 + 1e-9)
    att = jnp.concatenate([att0, att1], axis=1)
    out = (jnp.dot(att, wredT_ref[...], preferred_element_type=jnp.float32)
           + bred_ref[...] + hn_ref[...])
    m = jnp.mean(out, axis=1, keepdims=True)
    v = jnp.mean((out - m) ** 2, axis=1, keepdims=True)
    hs2 = (out - m) / jnp.sqrt(v + 1e-5) * g2_ref[...] + b2g_ref[...]
    x = jnp.dot(hs2, w1T_ref[...], preferred_element_type=jnp.float32) + b1_ref[...]
    x = jnp.where(x > 0, x, jnp.exp(jnp.where(x > 0, 0.0, x)) - 1.0)
    y = jnp.dot(x, w2T_ref[...], preferred_element_type=jnp.float32) + b2_ref[...]
    y = jnp.where(y > 0, y, jnp.exp(jnp.where(y > 0, 0.0, y)) - 1.0)
    out_ref[...] = y + hs2


def _tc2(a0, a1, d0, d1, hn, s4, wredT, bred, g2, b2g, w1T, b1, w2T, b2):
    grid = N // BN
    return pl.pallas_call(
        _tc2_body,
        grid=(grid,),
        in_specs=[
            pl.BlockSpec((1, BN, HALF), lambda i: (0, i, 0)),
            pl.BlockSpec((1, BN, HALF), lambda i: (1, i, 0)),
            pl.BlockSpec((1, BN, HPC), lambda i: (0, i, 0)),
            pl.BlockSpec((1, BN, HPC), lambda i: (1, i, 0)),
            pl.BlockSpec((BN, D), lambda i: (i, 0)),
            pl.BlockSpec((HPC, HALF), lambda i: (0, 0)),
            pl.BlockSpec((D, D), lambda i: (0, 0)),
            pl.BlockSpec((1, D), lambda i: (0, 0)),
            pl.BlockSpec((1, D), lambda i: (0, 0)),
            pl.BlockSpec((1, D), lambda i: (0, 0)),
            pl.BlockSpec((D, FF), lambda i: (0, 0)),
            pl.BlockSpec((1, FF), lambda i: (0, 0)),
            pl.BlockSpec((FF, D), lambda i: (0, 0)),
            pl.BlockSpec((1, D), lambda i: (0, 0)),
        ],
        out_specs=pl.BlockSpec((BN, D), lambda i: (i, 0)),
        out_shape=jax.ShapeDtypeStruct((N, D), jnp.float32),
    )(a0, a1, d0, d1, hn, s4, wredT, bred, g2, b2g, w1T, b1, w2T, b2)


def kernel(h, edge_index, Wfc, Wred, bred, ln1_g, ln1_b, ln2_g, ln2_b, W1, b1, W2, b2):
    src = edge_index[0]
    dst = edge_index[1]

    hn, ft2 = _tc1(h, Wfc.T, ln1_g.reshape(1, D), ln1_b.reshape(1, D))
    src2 = src * 2
    dst3 = dst.reshape(NS, NCHUNK, 1, CHUNK)
    zn = jnp.zeros((NP, HALF), jnp.float32)

    acc, den = _edge_sc(ft2, src2, dst3, zn)
    dp = den.reshape(NC, NP, HPC)

    s4 = jnp.repeat(jnp.eye(HPC, dtype=jnp.float32), DH, axis=1)

    out = _tc2(acc, acc, dp, dp, hn, s4, Wred.T,
               bred.reshape(1, D), ln2_g.reshape(1, D), ln2_b.reshape(1, D),
               W1.T, b1.reshape(1, FF), W2.T, b2.reshape(1, D))
    return out
